# Initial kernel scaffold; baseline (speedup 1.0000x reference)
import jax
import jax.numpy as jnp
from jax.experimental import pallas as pl


def kernel(emb, edge_index):
    raise NotImplementedError("write your pallas kernel here")


# speedup vs baseline: 9.8410x; 9.8410x over previous
import functools

import jax
import jax.numpy as jnp
from jax.experimental import pallas as pl
from jax.experimental.pallas import tpu as pltpu
from jax.experimental.pallas import tpu_sc as plsc

N = 10000
D = 128
E = 320000

NC = 2
NS = 16
NW = NC * NS
L = 16

CHUNK = 128
NCH = 2 * -(-E // (NW * 2 * CHUNK))
EPT = NCH * CHUNK
EPAD = NW * EPT
NPAD = -(-N // (NS * CHUNK)) * (NS * CHUNK)
NPT = NPAD // NS
_mesh = functools.partial(
    plsc.VectorSubcoreMesh,
    core_axis_name="c",
    subcore_axis_name="s",
    num_cores=NC,
    num_subcores=NS,
)


def _deg_body(colp, deg_out, colbuf, onesbuf, zbuf, sh_deg):
    cid = jax.lax.axis_index("c")
    sid = jax.lax.axis_index("s")
    wid = sid * NC + cid

    @pl.loop(0, CHUNK // L)
    def _(i):
        onesbuf[pl.ds(i * L, L)] = jnp.ones((L,), jnp.float32)

    @pl.loop(0, NPT // L)
    def _(i):
        zbuf[pl.ds(i * L, L)] = jnp.zeros((L,), jnp.float32)

    pltpu.sync_copy(zbuf, sh_deg.at[pl.ds(sid * NPT, NPT)])
    plsc.subcore_barrier()

    base = wid * EPT

    @pl.loop(0, NCH)
    def _(i):
        pltpu.sync_copy(colp.at[pl.ds(base + i * CHUNK, CHUNK)], colbuf)
        pltpu.sync_copy(onesbuf, sh_deg.at[colbuf], add=True)

    plsc.subcore_barrier()
    pltpu.sync_copy(
        sh_deg.at[pl.ds(sid * NPT, NPT)], deg_out.at[cid, pl.ds(sid * NPT, NPT)]
    )


def _deg_partials(colp):
    return pl.kernel(
        _deg_body,
        out_type=jax.ShapeDtypeStruct((NC, NPAD), jnp.float32),
        mesh=_mesh(),
        scratch_types=[
            pltpu.VMEM((CHUNK,), jnp.int32),
            pltpu.VMEM((CHUNK,), jnp.float32),
            pltpu.VMEM((NPT,), jnp.float32),
            pltpu.VMEM_SHARED((NPAD,), jnp.float32),
        ],
    )(colp)


def _scale_body(dp_ref, emb_ref, dis_ref, tmp_ref):
    deg = dp_ref[0] + dp_ref[1]
    dis = jnp.where(deg > 0.0, jax.lax.rsqrt(deg), 0.0)
    dis_ref[...] = dis
    tmp_ref[...] = dis[:N] * emb_ref[...]


def _scale(deg_partials, emb):
    return pl.pallas_call(
        _scale_body,
        out_shape=[
            jax.ShapeDtypeStruct((NPAD, 1), jnp.float32),
            jax.ShapeDtypeStruct((N, D), jnp.float32),
        ],
    )(deg_partials.reshape(NC, NPAD, 1), emb)


def _prop_body(tmp, rowp, colp, acc_out, ridx, cidx, rbuf, sh_acc, sem):
    cid = jax.lax.axis_index("c")
    sid = jax.lax.axis_index("s")
    wid = sid * NC + cid

    @pl.loop(0, CHUNK)
    def _(r):
        for c in range(D // L):
            rbuf[r, pl.ds(c * L, L)] = jnp.zeros((L,), jnp.float32)

    for k in range(NPT // CHUNK):
        pltpu.sync_copy(rbuf, sh_acc.at[pl.ds(sid * NPT + k * CHUNK, CHUNK)])
    plsc.subcore_barrier()

    base = wid * EPT

    @pl.loop(0, NCH)
    def _(i):
        off = base + i * CHUNK
        pltpu.sync_copy(rowp.at[pl.ds(off, CHUNK)], ridx)
        pltpu.sync_copy(colp.at[pl.ds(off, CHUNK)], cidx)
        pltpu.async_copy(tmp.at[ridx], rbuf, sem).wait()
        pltpu.sync_copy(rbuf, sh_acc.at[cidx], add=True)

    plsc.subcore_barrier()
    for k in range(NPT // CHUNK):
        r0 = sid * NPT + k * CHUNK
        pltpu.sync_copy(sh_acc.at[pl.ds(r0, CHUNK)], acc_out.at[cid, pl.ds(r0, CHUNK)])


def _propagate(tmp, rowp, colp):
    return pl.kernel(
        _prop_body,
        out_type=jax.ShapeDtypeStruct((NC, NPAD, D), jnp.float32),
        mesh=_mesh(),
        scratch_types=[
            pltpu.VMEM((CHUNK,), jnp.int32),
            pltpu.VMEM((CHUNK,), jnp.int32),
            pltpu.VMEM((CHUNK, D), jnp.float32),
            pltpu.VMEM_SHARED((NPAD, D), jnp.float32),
            pltpu.SemaphoreType.DMA,
        ],
    )(tmp, rowp, colp)


def _combine_body(ap_ref, dis_ref, out_ref):
    s = ap_ref[0, :N, :] + ap_ref[1, :N, :]
    out_ref[...] = dis_ref[:N] * s


def _combine(acc_partials, dis):
    return pl.pallas_call(
        _combine_body,
        out_shape=jax.ShapeDtypeStruct((N, D), jnp.float32),
    )(acc_partials, dis)


@jax.jit
def kernel(emb, edge_index):
    row = edge_index[0].astype(jnp.int32)
    col = edge_index[1].astype(jnp.int32)
    rowp = jnp.concatenate([row, jnp.zeros((EPAD - E,), jnp.int32)])
    colp = jnp.concatenate([col, jnp.full((EPAD - E,), NPAD - 1, jnp.int32)])

    dp = _deg_partials(colp)
    dis, tmp = _scale(dp, emb)
    acc = _propagate(tmp, rowp, colp)
    return _combine(acc, dis)


# speedup vs baseline: 12.7276x; 1.2933x over previous
import functools

import jax
import jax.numpy as jnp
from jax.experimental import pallas as pl
from jax.experimental.pallas import tpu as pltpu
from jax.experimental.pallas import tpu_sc as plsc

N = 10000
D = 128
E = 320000

NC = 2
NS = 16
NW = NC * NS
L = 16

CHUNK = 128
NCH = 2 * -(-E // (NW * 2 * CHUNK))
EPT = NCH * CHUNK
EPAD = NW * EPT
NPAD = -(-N // (NS * CHUNK)) * (NS * CHUNK)
NPT = NPAD // NS
_mesh = functools.partial(
    plsc.VectorSubcoreMesh,
    core_axis_name="c",
    subcore_axis_name="s",
    num_cores=NC,
    num_subcores=NS,
)


def _deg_body(colp, deg_out, cidx_all, onesbuf, zbuf, sh_deg):
    cid = jax.lax.axis_index("c")
    sid = jax.lax.axis_index("s")
    wid = sid * NC + cid

    @pl.loop(0, CHUNK // L)
    def _(i):
        onesbuf[pl.ds(i * L, L)] = jnp.ones((L,), jnp.float32)

    @pl.loop(0, NPT // L)
    def _(i):
        zbuf[pl.ds(i * L, L)] = jnp.zeros((L,), jnp.float32)

    pltpu.sync_copy(colp.at[wid], cidx_all)

    pltpu.sync_copy(zbuf, sh_deg.at[pl.ds(sid * NPT, NPT)])
    plsc.subcore_barrier()

    @pl.loop(0, NCH)
    def _(i):
        pltpu.sync_copy(onesbuf, sh_deg.at[cidx_all.at[i]], add=True)

    plsc.subcore_barrier()
    pltpu.sync_copy(
        sh_deg.at[pl.ds(sid * NPT, NPT)], deg_out.at[cid, pl.ds(sid * NPT, NPT)]
    )


def _deg_partials(colp):
    return pl.kernel(
        _deg_body,
        out_type=jax.ShapeDtypeStruct((NC, NPAD), jnp.float32),
        mesh=_mesh(),
        scratch_types=[
            pltpu.VMEM((NCH, CHUNK), jnp.int32),
            pltpu.VMEM((CHUNK,), jnp.float32),
            pltpu.VMEM((NPT,), jnp.float32),
            pltpu.VMEM_SHARED((NPAD,), jnp.float32),
        ],
    )(colp)


def _scale_body(dp_ref, emb_ref, dis_ref, tmp_ref):
    deg = dp_ref[0] + dp_ref[1]
    dis = jnp.where(deg > 0.0, jax.lax.rsqrt(deg), 0.0)
    dis_ref[...] = dis
    tmp_ref[...] = dis[:N] * emb_ref[...]


def _scale(deg_partials, emb):
    return pl.pallas_call(
        _scale_body,
        out_shape=[
            jax.ShapeDtypeStruct((NPAD, 1), jnp.float32),
            jax.ShapeDtypeStruct((N, D), jnp.float32),
        ],
    )(deg_partials.reshape(NC, NPAD, 1), emb)


NBUF = 2
NHALF = 2
HCH = NCH // NHALF


def _prop_body(tmp, rowp, colp, acc_out, ridx_all, cidx_all, rbufs, sems, sh_acc):
    cid = jax.lax.axis_index("c")
    sid = jax.lax.axis_index("s")
    wid = sid * NC + cid

    @pl.loop(0, CHUNK)
    def _(r):
        for c in range(D // L):
            rbufs[0][r, pl.ds(c * L, L)] = jnp.zeros((L,), jnp.float32)

    for k in range(NPT // CHUNK):
        pltpu.sync_copy(rbufs[0], sh_acc.at[pl.ds(sid * NPT + k * CHUNK, CHUNK)])
    plsc.subcore_barrier()

    for h in range(NHALF):
        pltpu.sync_copy(rowp.at[wid, pl.ds(h * HCH, HCH)], ridx_all)
        pltpu.sync_copy(colp.at[wid, pl.ds(h * HCH, HCH)], cidx_all)

        for b in range(NBUF):
            pltpu.async_copy(tmp.at[ridx_all.at[b]], rbufs[b], sems[b])

        @pl.loop(0, HCH, step=NBUF)
        def _(g):
            for b in range(NBUF):
                i = g + b
                pltpu.make_async_copy(tmp.at[ridx_all.at[i]], rbufs[b], sems[b]).wait()
                pltpu.sync_copy(rbufs[b], sh_acc.at[cidx_all.at[i]], add=True)

                @pl.when(i + NBUF < HCH)
                def _():
                    pltpu.async_copy(tmp.at[ridx_all.at[i + NBUF]], rbufs[b], sems[b])

    plsc.subcore_barrier()
    for k in range(NPT // CHUNK):
        r0 = sid * NPT + k * CHUNK
        pltpu.sync_copy(sh_acc.at[pl.ds(r0, CHUNK)], acc_out.at[cid, pl.ds(r0, CHUNK)])


def _propagate(tmp, rowp, colp):
    return pl.kernel(
        _prop_body,
        out_type=jax.ShapeDtypeStruct((NC, NPAD, D), jnp.float32),
        mesh=_mesh(),
        scratch_types=[
            pltpu.VMEM((HCH, CHUNK), jnp.int32),
            pltpu.VMEM((HCH, CHUNK), jnp.int32),
            [pltpu.VMEM((CHUNK, D), jnp.float32) for _ in range(NBUF)],
            [pltpu.SemaphoreType.DMA for _ in range(NBUF)],
            pltpu.VMEM_SHARED((NPAD, D), jnp.float32),
        ],
    )(tmp, rowp, colp)


def _combine_body(ap_ref, dis_ref, out_ref):
    s = ap_ref[0, :N, :] + ap_ref[1, :N, :]
    out_ref[...] = dis_ref[:N] * s


def _combine(acc_partials, dis):
    return pl.pallas_call(
        _combine_body,
        out_shape=jax.ShapeDtypeStruct((N, D), jnp.float32),
    )(acc_partials, dis)


@jax.jit
def kernel(emb, edge_index):
    row = edge_index[0].astype(jnp.int32)
    col = edge_index[1].astype(jnp.int32)
    rowp = jnp.concatenate([row, jnp.zeros((EPAD - E,), jnp.int32)])
    colp = jnp.concatenate([col, jnp.full((EPAD - E,), NPAD - 1, jnp.int32)])
    rowp = rowp.reshape(NW, NCH, CHUNK)
    colp = colp.reshape(NW, NCH, CHUNK)

    dp = _deg_partials(colp)
    dis, tmp = _scale(dp, emb)
    acc = _propagate(tmp, rowp, colp)
    return _combine(acc, dis)
